# baseline (speedup 1.0000x reference)
import functools

import jax
import jax.numpy as jnp
from jax import lax
from jax.experimental import pallas as pl
from jax.experimental.pallas import tpu as pltpu
from jax.experimental.pallas import tpu_sc as plsc

_CHUNK = 128


def _gather_fn(V, B, D, nc, ns):
    nw = nc * ns
    b_per_w = B // nw
    n_chunks = b_per_w // _CHUNK
    mesh = plsc.VectorSubcoreMesh(core_axis_name="c", subcore_axis_name="s")

    @functools.partial(
        pl.kernel,
        mesh=mesh,
        out_type=jax.ShapeDtypeStruct((B, D), jnp.float32),
        scratch_types=[
            pltpu.VMEM((n_chunks, _CHUNK), jnp.int32),
            pltpu.VMEM((n_chunks, _CHUNK, D), jnp.float32),
            pltpu.VMEM_SHARED((V, D), jnp.float32),
            pltpu.SemaphoreType.DMA,
            pltpu.SemaphoreType.DMA,
            pltpu.SemaphoreType.DMA,
        ],
    )
    def k(table_hbm, idx_hbm, out_hbm, idx_v, rows_v, table_sp,
          sem_h, sem_g, sem_w):
        cid = lax.axis_index("c")
        sid = lax.axis_index("s")
        wid = sid * nc + cid
        base = wid * b_per_w

        pltpu.sync_copy(idx_hbm.at[wid], idx_v)
        g0 = pltpu.async_copy(table_hbm.at[idx_v.at[0]], rows_v.at[0], sem_h)

        @pl.when(sid == 0)
        def _():
            pltpu.sync_copy(table_hbm, table_sp)

        plsc.subcore_barrier()

        gathers = [g0] + [
            pltpu.async_copy(table_sp.at[idx_v.at[i]], rows_v.at[i], sem_g)
            for i in range(1, n_chunks)
        ]
        writes = []
        for i in range(n_chunks):
            gathers[i].wait()
            writes.append(
                pltpu.async_copy(
                    rows_v.at[i],
                    out_hbm.at[pl.ds(base + i * _CHUNK, _CHUNK)],
                    sem_w,
                )
            )
        for w in writes:
            w.wait()

    return k


def kernel(seq, state_idx, embedding_table):
    V, D = embedding_table.shape
    B = state_idx.shape[0]
    info = plsc.get_sparse_core_info()
    nc, ns = info.num_cores, info.num_subcores
    idx = state_idx.reshape(nc * ns, B // (nc * ns) // _CHUNK, _CHUNK)
    return _gather_fn(V, B, D, nc, ns)(embedding_table, idx)


# speedup vs baseline: 1.0213x; 1.0213x over previous
import functools

import jax
import jax.numpy as jnp
from jax import lax
from jax.experimental import pallas as pl
from jax.experimental.pallas import tpu as pltpu
from jax.experimental.pallas import tpu_sc as plsc

_CHUNK = 128


def _gather_fn(V, B, D, nc, ns):
    nw = nc * ns
    b_per_w = B // nw
    n_chunks = b_per_w // _CHUNK
    n_stagers = 5
    v_per_stager = V // n_stagers
    mesh = plsc.VectorSubcoreMesh(core_axis_name="c", subcore_axis_name="s")

    @functools.partial(
        pl.kernel,
        mesh=mesh,
        out_type=jax.ShapeDtypeStruct((B, D), jnp.float32),
        scratch_types=[
            pltpu.VMEM((n_chunks, _CHUNK), jnp.int32),
            pltpu.VMEM((n_chunks, _CHUNK, D), jnp.float32),
            pltpu.VMEM_SHARED((V, D), jnp.float32),
            pltpu.SemaphoreType.DMA,
            pltpu.SemaphoreType.DMA,
        ],
    )
    def k(table_hbm, idx_hbm, out_hbm, idx_v, rows_v, table_sp, sem_g, sem_w):
        cid = lax.axis_index("c")
        sid = lax.axis_index("s")
        wid = sid * nc + cid
        base = wid * b_per_w

        @pl.when(sid < n_stagers)
        def _():
            r0 = sid * v_per_stager
            pltpu.sync_copy(
                table_hbm.at[pl.ds(r0, v_per_stager)],
                table_sp.at[pl.ds(r0, v_per_stager)],
            )

        pltpu.sync_copy(idx_hbm.at[wid], idx_v)
        plsc.subcore_barrier()

        gathers = [
            pltpu.async_copy(table_sp.at[idx_v.at[i]], rows_v.at[i], sem_g)
            for i in range(n_chunks)
        ]
        writes = []
        for i in range(n_chunks):
            gathers[i].wait()
            writes.append(
                pltpu.async_copy(
                    rows_v.at[i],
                    out_hbm.at[pl.ds(base + i * _CHUNK, _CHUNK)],
                    sem_w,
                )
            )
        for w in writes:
            w.wait()

    return k


def kernel(seq, state_idx, embedding_table):
    V, D = embedding_table.shape
    B = state_idx.shape[0]
    info = plsc.get_sparse_core_info()
    nc, ns = info.num_cores, info.num_subcores
    idx = state_idx.reshape(nc * ns, B // (nc * ns) // _CHUNK, _CHUNK)
    return _gather_fn(V, B, D, nc, ns)(embedding_table, idx)
